# baseline (speedup 1.0000x reference)
import functools

import jax
import jax.numpy as jnp
from jax import lax
from jax.experimental import pallas as pl
from jax.experimental.pallas import tpu as pltpu
from jax.experimental.pallas import tpu_sc as plsc

VOCAB = 100000
D = 128
B = 16384
NSLOT = 5
PCOL = 8

NC = 2
NS = 16
NW = NC * NS
BPW = B // NW
SUB = BPW // 128
NROW = NSLOT * SUB


def _tc_project_body(g_ref, w_ref, p_ref):
    p_ref[...] = jnp.dot(
        g_ref[...].astype(jnp.bfloat16),
        w_ref[...].astype(jnp.bfloat16),
        preferred_element_type=jnp.float32,
    )


def _tc_project(g_feats, w_pad):
    rows = 10000
    grid = VOCAB // rows
    return pl.pallas_call(
        _tc_project_body,
        grid=(grid,),
        in_specs=[
            pl.BlockSpec((rows, D), lambda i: (i, 0)),
            pl.BlockSpec((D, PCOL), lambda i: (0, 0)),
        ],
        out_specs=pl.BlockSpec((rows, PCOL), lambda i: (i, 0)),
        out_shape=jax.ShapeDtypeStruct((VOCAB, PCOL), jnp.float32),
    )(g_feats, w_pad)


def _sc_gather(p_flat, samples_flat, bias16):
    mesh = plsc.VectorSubcoreMesh(core_axis_name="c", subcore_axis_name="s")

    @functools.partial(
        pl.kernel,
        mesh=mesh,
        out_type=jax.ShapeDtypeStruct((B,), jnp.float32),
        scratch_types=[
            pltpu.VMEM((NROW, 128), jnp.int32),
            pltpu.VMEM((NROW * 128,), jnp.int32),
            pltpu.VMEM((NROW, 128), jnp.int32),
            pltpu.VMEM((NROW * 128,), jnp.float32),
            pltpu.VMEM((BPW,), jnp.float32),
            pltpu.VMEM((16,), jnp.float32),
            pltpu.SemaphoreType.DMA,
        ],
    )
    def sc_k(pflat_hbm, sflat_hbm, bias_hbm, out_hbm,
             cidx, svT, fidx, gbuf, acc, bv, sem):
        wid = lax.axis_index("s") * NC + lax.axis_index("c")
        base = wid * BPW
        pltpu.sync_copy(bias_hbm, bv)
        io5 = lax.iota(jnp.int32, 16) * NSLOT
        h1 = []
        for r in range(NROW):
            j, s = r // SUB, r % SUB
            for c in range(8):
                t = s * 8 + c
                cidx[r, pl.ds(c * 16, 16)] = io5 + (base * NSLOT + j + 80 * t)
            h1.append(
                pltpu.async_copy(
                    sflat_hbm.at[cidx.at[r]],
                    svT.at[pl.ds(r * 128, 128)],
                    sem,
                )
            )
        for h in h1:
            h.wait()
        h2 = []
        for r in range(NROW):
            j = r // SUB
            for c in range(8):
                ids = svT[pl.ds(r * 128 + c * 16, 16)]
                fidx[r, pl.ds(c * 16, 16)] = ids * PCOL + j
            h2.append(
                pltpu.async_copy(
                    pflat_hbm.at[fidx.at[r]],
                    gbuf.at[pl.ds(r * 128, 128)],
                    sem,
                )
            )
        for h in h2:
            h.wait()
        bias_v = bv[...]
        for c in range(BPW // 16):
            tot = bias_v
            for j in range(NSLOT):
                tot = tot + gbuf[pl.ds(j * BPW + c * 16, 16)]
            acc[pl.ds(c * 16, 16)] = tot
        pltpu.sync_copy(acc, out_hbm.at[pl.ds(base, BPW)])

    return sc_k(p_flat, samples_flat, bias16)


def kernel(g_feats, samples, W, b):
    w_pad = jnp.zeros((D, PCOL), jnp.float32).at[:, :NSLOT].set(
        W.reshape(NSLOT, D).T
    )
    p = _tc_project(g_feats, w_pad)
    p_flat = p.reshape(-1)
    samples_flat = samples.reshape(-1)
    bias16 = jnp.full((16,), b[0], jnp.float32)
    out_flat = _sc_gather(p_flat, samples_flat, bias16)
    return out_flat.reshape(B, 1)


# speedup vs baseline: 1.2092x; 1.2092x over previous
import functools

import jax
import jax.numpy as jnp
from jax import lax
from jax.experimental import pallas as pl
from jax.experimental.pallas import tpu as pltpu
from jax.experimental.pallas import tpu_sc as plsc

VOCAB = 100000
D = 128
B = 16384
NSLOT = 5
PCOL = 8

NC = 2
NS = 16
NW = NC * NS
BPW = B // NW
SUB = BPW // 128


def _tc_project_body(g_ref, w_ref, p_ref):
    p_ref[...] = jnp.dot(
        g_ref[...].astype(jnp.bfloat16),
        w_ref[...].astype(jnp.bfloat16),
        preferred_element_type=jnp.float32,
    )


def _tc_project(g_feats, w_pad):
    rows = 20000
    grid = VOCAB // rows
    return pl.pallas_call(
        _tc_project_body,
        grid=(grid,),
        in_specs=[
            pl.BlockSpec((rows, D), lambda i: (i, 0)),
            pl.BlockSpec((D, PCOL), lambda i: (0, 0)),
        ],
        out_specs=pl.BlockSpec((rows, PCOL), lambda i: (i, 0)),
        out_shape=jax.ShapeDtypeStruct((VOCAB, PCOL), jnp.float32),
    )(g_feats, w_pad)


def _sc_gather(p_flat, samples_t, bias16):
    mesh = plsc.VectorSubcoreMesh(core_axis_name="c", subcore_axis_name="s")

    @functools.partial(
        pl.kernel,
        mesh=mesh,
        out_type=jax.ShapeDtypeStruct((B,), jnp.float32),
        scratch_types=[
            pltpu.VMEM((NSLOT, BPW), jnp.int32),
            pltpu.VMEM((NSLOT * SUB, 128), jnp.int32),
            pltpu.VMEM((NSLOT * SUB, 128), jnp.float32),
            pltpu.VMEM((BPW,), jnp.float32),
            pltpu.VMEM((16,), jnp.float32),
            pltpu.SemaphoreType.DMA,
        ],
    )
    def sc_k(pflat_hbm, st_hbm, bias_hbm, out_hbm, sv, fidx, gbuf, acc, bv, sem):
        wid = lax.axis_index("s") * NC + lax.axis_index("c")
        base = wid * BPW
        pltpu.sync_copy(st_hbm.at[:, pl.ds(base, BPW)], sv)
        pltpu.sync_copy(bias_hbm, bv)
        handles = []
        for j in range(NSLOT):
            for s in range(SUB):
                row = j * SUB + s
                for c in range(8):
                    ids = sv[j, pl.ds(s * 128 + c * 16, 16)]
                    fidx[row, pl.ds(c * 16, 16)] = ids * PCOL + j
                handles.append(
                    pltpu.async_copy(pflat_hbm.at[fidx.at[row]], gbuf.at[row], sem)
                )
        for h in handles:
            h.wait()
        bias_v = bv[...]
        for c in range(BPW // 16):
            s = c // 8
            off = (c % 8) * 16
            tot = bias_v
            for j in range(NSLOT):
                tot = tot + gbuf[j * SUB + s, pl.ds(off, 16)]
            acc[pl.ds(c * 16, 16)] = tot
        pltpu.sync_copy(acc, out_hbm.at[pl.ds(base, BPW)])

    return sc_k(p_flat, samples_t, bias16)


def kernel(g_feats, samples, W, b):
    w_pad = jnp.zeros((D, PCOL), jnp.float32).at[:, :NSLOT].set(
        W.reshape(NSLOT, D).T
    )
    p = _tc_project(g_feats, w_pad)
    p_flat = p.reshape(-1)
    samples_t = samples.T
    bias16 = jnp.full((16,), b[0], jnp.float32)
    out_flat = _sc_gather(p_flat, samples_t, bias16)
    return out_flat.reshape(B, 1)
